# baseline (speedup 1.0000x reference)
import functools

import jax
import jax.numpy as jnp
from jax import lax
from jax.experimental import pallas as pl
from jax.experimental.pallas import tpu as pltpu
from jax.experimental.pallas import tpu_sc as plsc

T = 4096
D = 2048
E = 8
TM = 512
MB = 256
R = 2 * T + E * MB
NT = R // MB
TME = 512

NW = 32
TW = T // NW
CH = 16
NCH = TW // CH
GW = 2 * T // NW
NGC = GW // CH

_MESH = plsc.VectorSubcoreMesh(core_axis_name="c", subcore_axis_name="s")
D2 = D // 2


def _as_i32(a):
    return jax.lax.bitcast_convert_type(
        a.reshape(*a.shape[:-1], a.shape[-1] // 2, 2), jnp.int32)


def _as_bf16(a):
    b = jax.lax.bitcast_convert_type(a, jnp.bfloat16)
    return b.reshape(*a.shape[:-1], a.shape[-1] * 2)


@functools.partial(
    pl.kernel, mesh=_MESH,
    out_type=jax.ShapeDtypeStruct((R, D2), jnp.int32),
    scratch_types=[
        pltpu.VMEM((CH, D2), jnp.int32),
        pltpu.VMEM((NCH, CH), jnp.int32),
        pltpu.VMEM((NCH, CH), jnp.int32),
        pltpu.SemaphoreType.DMA,
    ],
)
def _dispatch(x_hbm, pos0_hbm, pos1_hbm, xs_hbm, xrows, p0_v, p1_v, sem):
    wid = lax.axis_index("s") * 2 + lax.axis_index("c")
    base = wid * TW
    pltpu.sync_copy(pos0_hbm.at[wid], p0_v)
    pltpu.sync_copy(pos1_hbm.at[wid], p1_v)
    for c in range(NCH):
        pltpu.sync_copy(x_hbm.at[pl.ds(base + c * CH, CH)], xrows)
        cp0 = pltpu.async_copy(xrows, xs_hbm.at[p0_v.at[c]], sem)
        cp1 = pltpu.async_copy(xrows, xs_hbm.at[p1_v.at[c]], sem)
        cp0.wait()
        cp1.wait()


@functools.partial(
    pl.kernel, mesh=_MESH,
    out_type=jax.ShapeDtypeStruct((2 * T, D2), jnp.int32),
    scratch_types=[
        pltpu.VMEM((CH, D2), jnp.int32),
        pltpu.VMEM((NGC, CH), jnp.int32),
        pltpu.SemaphoreType.DMA,
    ],
)
def _collect(ys_hbm, posf_hbm, out_hbm, rows_v, idx_v, sem):
    wid = lax.axis_index("s") * 2 + lax.axis_index("c")
    base = wid * GW
    pltpu.sync_copy(posf_hbm.at[wid], idx_v)
    for c in range(NGC):
        pltpu.async_copy(ys_hbm.at[idx_v.at[c]], rows_v, sem).wait()
        pltpu.sync_copy(rows_v, out_hbm.at[pl.ds(base + c * CH, CH)])


def _a1_body(x_ref, gw_ref, gb_ref, lt_ref,
             g_ref, e01_ref, w01_ref, r01_ref, cnt_out_ref, cnt_ref):
    i = pl.program_id(0)

    @pl.when(i == 0)
    def _():
        cnt_ref[...] = jnp.zeros((1, E), jnp.float32)

    xf = x_ref[...]
    gs = jax.lax.dot_general(
        xf, gw_ref[...], (((1,), (1,)), ((), ())),
        preferred_element_type=jnp.float32,
    ) + gb_ref[...]
    lanes = jax.lax.broadcasted_iota(jnp.int32, (TM, E), 1)
    i0 = jnp.argmax(gs, axis=1)[:, None]
    oh0 = lanes == i0
    masked = jnp.where(oh0, -jnp.inf, gs)
    i1 = jnp.argmax(masked, axis=1)[:, None]
    oh1 = lanes == i1
    m0 = jnp.max(gs, axis=1)[:, None]
    m1 = jnp.max(masked, axis=1)[:, None]
    e1 = jnp.exp(m1 - m0)
    w0 = 1.0 / (1.0 + e1)
    w1 = e1 / (1.0 + e1)
    g_ref[...] = jnp.where(oh0, w0, 0.0) + jnp.where(oh1, w1, 0.0)
    sel = (oh0 | oh1).astype(jnp.float32)
    rank_local = jax.lax.dot_general(
        lt_ref[...], sel.astype(jnp.bfloat16), (((1,), (0,)), ((), ())),
        preferred_element_type=jnp.float32,
    )
    rank = rank_local + cnt_ref[...]
    cnt_ref[...] += jnp.sum(sel, axis=0, keepdims=True)
    cnt_out_ref[...] = cnt_ref[...]
    e01_ref[...] = jnp.concatenate([i0, i1], axis=1).astype(jnp.int32)
    w01_ref[...] = jnp.concatenate([w0, w1], axis=1)
    r0 = jnp.sum(jnp.where(oh0, rank, 0.0), axis=1, keepdims=True)
    r1 = jnp.sum(jnp.where(oh1, rank, 0.0), axis=1, keepdims=True)
    r01_ref[...] = jnp.concatenate([r0, r1], axis=1).astype(jnp.int32)


def _a2_body(cnt_ref, e01_ref, r01_ref, pos_ref, eid_ref):
    counts = cnt_ref[...]
    npad = jnp.ceil(counts / MB) * MB
    ri = jax.lax.broadcasted_iota(jnp.int32, (E, E), 0)
    ci = jax.lax.broadcasted_iota(jnp.int32, (E, E), 1)
    m = (ri < ci).astype(jnp.float32)
    start = jax.lax.dot_general(
        npad, m, (((1,), (0,)), ((), ())),
        preferred_element_type=jnp.float32,
        precision=jax.lax.Precision.HIGHEST,
    )
    e01 = e01_ref[...]
    base = jnp.zeros(e01.shape, jnp.float32)
    for e in range(E):
        base += jnp.where(e01 == e, start[0, e], 0.0)
    pos_ref[...] = (base + r01_ref[...].astype(jnp.float32)).astype(jnp.int32)
    trow = jax.lax.broadcasted_iota(jnp.int32, (1, NT), 1).astype(jnp.float32) * MB
    eid = jnp.zeros((1, NT), jnp.int32)
    for e in range(E):
        inside = (trow >= start[0, e]) & (trow < start[0, e] + npad[0, e])
        eid = jnp.where(inside, e, eid)
    eid_ref[...] = eid


def _c_body(eid_ref, xs_ref, w_ref, ys_ref):
    ys_ref[...] = jax.lax.dot_general(
        xs_ref[...], w_ref[0], (((1,), (1,)), ((), ())),
        preferred_element_type=jnp.float32,
    ).astype(jnp.bfloat16)


def _e_body(x_ref, sw_ref, sb_ref, eb_ref, g_ref, w01_ref, y0_ref, y1_ref,
            out_ref):
    xf = x_ref[...]
    mm = jax.lax.dot_general(
        xf.astype(jnp.bfloat16), sw_ref[...], (((1,), (1,)), ((), ())),
        preferred_element_type=jnp.float32,
    )
    bterm = jax.lax.dot_general(
        g_ref[...], eb_ref[...], (((1,), (0,)), ((), ())),
        preferred_element_type=jnp.float32,
    )
    w01 = w01_ref[...]
    acc = (xf + mm + sb_ref[...] + bterm
           + w01[:, 0:1] * y0_ref[0].astype(jnp.float32)
           + w01[:, 1:2] * y1_ref[0].astype(jnp.float32))
    out_ref[...] = jnp.maximum(acc, 0.0)


def _gating(x, gate_W, gb):
    lt = jnp.tril(jnp.ones((TM, TM), jnp.bfloat16), -1)
    return pl.pallas_call(
        _a1_body,
        grid=(T // TM,),
        in_specs=[
            pl.BlockSpec((TM, D), lambda i: (i, 0)),
            pl.BlockSpec((E, D), lambda i: (0, 0)),
            pl.BlockSpec((1, E), lambda i: (0, 0)),
            pl.BlockSpec((TM, TM), lambda i: (0, 0)),
        ],
        out_specs=[
            pl.BlockSpec((TM, E), lambda i: (i, 0)),
            pl.BlockSpec((TM, 2), lambda i: (i, 0)),
            pl.BlockSpec((TM, 2), lambda i: (i, 0)),
            pl.BlockSpec((TM, 2), lambda i: (i, 0)),
            pl.BlockSpec((1, E), lambda i: (0, 0)),
        ],
        out_shape=[
            jax.ShapeDtypeStruct((T, E), jnp.float32),
            jax.ShapeDtypeStruct((T, 2), jnp.int32),
            jax.ShapeDtypeStruct((T, 2), jnp.float32),
            jax.ShapeDtypeStruct((T, 2), jnp.int32),
            jax.ShapeDtypeStruct((1, E), jnp.float32),
        ],
        scratch_shapes=[pltpu.VMEM((1, E), jnp.float32)],
        compiler_params=pltpu.CompilerParams(
            dimension_semantics=("arbitrary",),
        ),
    )(x, gate_W, gb, lt)


def _positions(counts, e01, r01):
    return pl.pallas_call(
        _a2_body,
        grid=(1,),
        in_specs=[
            pl.BlockSpec((1, E), lambda i: (0, 0)),
            pl.BlockSpec((T, 2), lambda i: (0, 0)),
            pl.BlockSpec((T, 2), lambda i: (0, 0)),
        ],
        out_specs=[
            pl.BlockSpec((T, 2), lambda i: (0, 0)),
            pl.BlockSpec((1, NT), lambda i: (0, 0)),
        ],
        out_shape=[
            jax.ShapeDtypeStruct((T, 2), jnp.int32),
            jax.ShapeDtypeStruct((1, NT), jnp.int32),
        ],
    )(counts, e01, r01)


def _grouped_matmul(xs, W_bf, eid):
    grid_spec = pltpu.PrefetchScalarGridSpec(
        num_scalar_prefetch=1,
        grid=(NT,),
        in_specs=[
            pl.BlockSpec((MB, D), lambda n, eid: (n, 0)),
            pl.BlockSpec((1, D, D), lambda n, eid: (eid[n], 0, 0)),
        ],
        out_specs=pl.BlockSpec((MB, D), lambda n, eid: (n, 0)),
    )
    return pl.pallas_call(
        _c_body,
        grid_spec=grid_spec,
        out_shape=jax.ShapeDtypeStruct((R, D), jnp.bfloat16),
        compiler_params=pltpu.CompilerParams(
            dimension_semantics=("arbitrary",),
        ),
    )(eid, xs, W_bf)


def _combine(x, sw_bf, sb, eb, g, w01, y0, y1):
    return pl.pallas_call(
        _e_body,
        grid=(T // TME,),
        in_specs=[
            pl.BlockSpec((TME, D), lambda i: (i, 0)),
            pl.BlockSpec((D, D), lambda i: (0, 0)),
            pl.BlockSpec((1, D), lambda i: (0, 0)),
            pl.BlockSpec((E, D), lambda i: (0, 0)),
            pl.BlockSpec((TME, E), lambda i: (i, 0)),
            pl.BlockSpec((TME, 2), lambda i: (i, 0)),
            pl.BlockSpec((1, TME, D), lambda i: (0, i, 0)),
            pl.BlockSpec((1, TME, D), lambda i: (1, i, 0)),
        ],
        out_specs=pl.BlockSpec((TME, D), lambda i: (i, 0)),
        out_shape=jax.ShapeDtypeStruct((T, D), jnp.float32),
        compiler_params=pltpu.CompilerParams(
            dimension_semantics=("parallel",),
        ),
    )(x, sw_bf, sb, eb, g, w01, y0, y1)


def kernel(x, shared_W, shared_b, gate_W, gate_b, gate_bias, expert_W, expert_b):
    gb = (gate_b + gate_bias).reshape(1, E)
    g, e01, w01, r01, counts = _gating(x, gate_W, gb)
    pos, eid2 = _positions(counts, e01, r01)
    eid = eid2.reshape(NT)

    pos0_r = pos[:, 0].reshape(NW, NCH, CH)
    pos1_r = pos[:, 1].reshape(NW, NCH, CH)
    xs_i = _dispatch(_as_i32(x.astype(jnp.bfloat16)), pos0_r, pos1_r)

    ys = _grouped_matmul(_as_bf16(xs_i), expert_W.astype(jnp.bfloat16), eid)

    posf = jnp.concatenate([pos[:, 0], pos[:, 1]]).reshape(NW, NGC, CH)
    ysg = _as_bf16(_collect(_as_i32(ys), posf)).reshape(2, T, D)

    return _combine(x, shared_W.astype(jnp.bfloat16), shared_b.reshape(1, D),
                    expert_b, g, w01, ysg, ysg)


# speedup vs baseline: 4.3436x; 4.3436x over previous
import functools

import jax
import jax.numpy as jnp
from jax import lax
from jax.experimental import pallas as pl
from jax.experimental.pallas import tpu as pltpu
from jax.experimental.pallas import tpu_sc as plsc

T = 4096
D = 2048
E = 8
TM = 512
MB = 256
R = 2 * T + E * MB
NT = R // MB
TME = 512

NW = 32
TW = T // NW
CH = 16
NCH = TW // CH
GW = 2 * T // NW
NGC = GW // CH

_MESH = plsc.VectorSubcoreMesh(core_axis_name="c", subcore_axis_name="s")


@functools.partial(
    pl.kernel, mesh=_MESH,
    out_type=jax.ShapeDtypeStruct((R, D), jnp.float32),
    scratch_types=[
        pltpu.VMEM((CH, D), jnp.float32),
        pltpu.VMEM((CH, D), jnp.float32),
        pltpu.VMEM((NCH, CH), jnp.int32),
        pltpu.VMEM((NCH, CH), jnp.int32),
        pltpu.SemaphoreType.DMA,
        pltpu.SemaphoreType.DMA,
    ],
)
def _dispatch(x_hbm, pos0_hbm, pos1_hbm, xs_hbm, xa, xb, p0_v, p1_v,
              ld_sem, st_sem):
    wid = lax.axis_index("s") * 2 + lax.axis_index("c")
    base = wid * TW
    pltpu.sync_copy(pos0_hbm.at[wid], p0_v)
    pltpu.sync_copy(pos1_hbm.at[wid], p1_v)
    bufs = [xa, xb]
    lds = [pltpu.async_copy(x_hbm.at[pl.ds(base, CH)], xa, ld_sem), None]
    scat = []
    for c in range(NCH):
        cur = bufs[c % 2]
        lds[c % 2].wait()
        scat.append(
            (pltpu.async_copy(cur, xs_hbm.at[p0_v.at[c]], st_sem),
             pltpu.async_copy(cur, xs_hbm.at[p1_v.at[c]], st_sem)))
        if c + 1 < NCH:
            if c >= 1:
                scat[c - 1][0].wait()
                scat[c - 1][1].wait()
            lds[(c + 1) % 2] = pltpu.async_copy(
                x_hbm.at[pl.ds(base + (c + 1) * CH, CH)],
                bufs[(c + 1) % 2], ld_sem)
    for s0, s1 in scat[NCH - 2:]:
        s0.wait()
        s1.wait()


@functools.partial(
    pl.kernel, mesh=_MESH,
    out_type=jax.ShapeDtypeStruct((2 * T, D), jnp.float32),
    scratch_types=[
        pltpu.VMEM((CH, D), jnp.float32),
        pltpu.VMEM((CH, D), jnp.float32),
        pltpu.VMEM((NGC, CH), jnp.int32),
        pltpu.SemaphoreType.DMA,
        pltpu.SemaphoreType.DMA,
    ],
)
def _collect(ys_hbm, posf_hbm, out_hbm, ra, rb, idx_v, g_sem, w_sem):
    wid = lax.axis_index("s") * 2 + lax.axis_index("c")
    base = wid * GW
    pltpu.sync_copy(posf_hbm.at[wid], idx_v)
    bufs = [ra, rb]
    gs = [pltpu.async_copy(ys_hbm.at[idx_v.at[0]], ra, g_sem), None]
    wrs = []
    for c in range(NGC):
        cur = bufs[c % 2]
        gs[c % 2].wait()
        wrs.append(pltpu.async_copy(
            cur, out_hbm.at[pl.ds(base + c * CH, CH)], w_sem))
        if c + 1 < NGC:
            if c >= 1:
                wrs[c - 1].wait()
            gs[(c + 1) % 2] = pltpu.async_copy(
                ys_hbm.at[idx_v.at[c + 1]], bufs[(c + 1) % 2], g_sem)
    for w in wrs[NGC - 2:]:
        w.wait()


def _a1_body(x_ref, gw_ref, gb_ref, lt_ref,
             g_ref, e01_ref, w01_ref, r01_ref, cnt_out_ref, cnt_ref):
    i = pl.program_id(0)

    @pl.when(i == 0)
    def _():
        cnt_ref[...] = jnp.zeros((1, E), jnp.float32)

    xf = x_ref[...]
    gs = jax.lax.dot_general(
        xf, gw_ref[...], (((1,), (1,)), ((), ())),
        preferred_element_type=jnp.float32,
    ) + gb_ref[...]
    lanes = jax.lax.broadcasted_iota(jnp.int32, (TM, E), 1)
    i0 = jnp.argmax(gs, axis=1)[:, None]
    oh0 = lanes == i0
    masked = jnp.where(oh0, -jnp.inf, gs)
    i1 = jnp.argmax(masked, axis=1)[:, None]
    oh1 = lanes == i1
    m0 = jnp.max(gs, axis=1)[:, None]
    m1 = jnp.max(masked, axis=1)[:, None]
    e1 = jnp.exp(m1 - m0)
    w0 = 1.0 / (1.0 + e1)
    w1 = e1 / (1.0 + e1)
    g_ref[...] = jnp.where(oh0, w0, 0.0) + jnp.where(oh1, w1, 0.0)
    sel = (oh0 | oh1).astype(jnp.float32)
    rank_local = jax.lax.dot_general(
        lt_ref[...], sel.astype(jnp.bfloat16), (((1,), (0,)), ((), ())),
        preferred_element_type=jnp.float32,
    )
    rank = rank_local + cnt_ref[...]
    cnt_ref[...] += jnp.sum(sel, axis=0, keepdims=True)
    cnt_out_ref[...] = cnt_ref[...]
    e01_ref[...] = jnp.concatenate([i0, i1], axis=1).astype(jnp.int32)
    w01_ref[...] = jnp.concatenate([w0, w1], axis=1)
    r0 = jnp.sum(jnp.where(oh0, rank, 0.0), axis=1, keepdims=True)
    r1 = jnp.sum(jnp.where(oh1, rank, 0.0), axis=1, keepdims=True)
    r01_ref[...] = jnp.concatenate([r0, r1], axis=1).astype(jnp.int32)


def _a2_body(cnt_ref, e01_ref, r01_ref, pos_ref, eid_ref):
    counts = cnt_ref[...]
    npad = jnp.ceil(counts / MB) * MB
    ri = jax.lax.broadcasted_iota(jnp.int32, (E, E), 0)
    ci = jax.lax.broadcasted_iota(jnp.int32, (E, E), 1)
    m = (ri < ci).astype(jnp.float32)
    start = jax.lax.dot_general(
        npad, m, (((1,), (0,)), ((), ())),
        preferred_element_type=jnp.float32,
        precision=jax.lax.Precision.HIGHEST,
    )
    e01 = e01_ref[...]
    base = jnp.zeros(e01.shape, jnp.float32)
    for e in range(E):
        base += jnp.where(e01 == e, start[0, e], 0.0)
    pos_ref[...] = (base + r01_ref[...].astype(jnp.float32)).astype(jnp.int32)
    trow = jax.lax.broadcasted_iota(jnp.int32, (1, NT), 1).astype(jnp.float32) * MB
    eid = jnp.zeros((1, NT), jnp.int32)
    for e in range(E):
        inside = (trow >= start[0, e]) & (trow < start[0, e] + npad[0, e])
        eid = jnp.where(inside, e, eid)
    eid_ref[...] = eid


def _c_body(eid_ref, xs_ref, w_ref, ys_ref):
    ys_ref[...] = jax.lax.dot_general(
        xs_ref[...].astype(jnp.bfloat16), w_ref[0], (((1,), (1,)), ((), ())),
        preferred_element_type=jnp.float32,
    )


def _e_body(x_ref, sw_ref, sb_ref, eb_ref, g_ref, w01_ref, y0_ref, y1_ref,
            out_ref):
    xf = x_ref[...]
    mm = jax.lax.dot_general(
        xf.astype(jnp.bfloat16), sw_ref[...], (((1,), (1,)), ((), ())),
        preferred_element_type=jnp.float32,
    )
    bterm = jax.lax.dot_general(
        g_ref[...], eb_ref[...], (((1,), (0,)), ((), ())),
        preferred_element_type=jnp.float32,
    )
    w01 = w01_ref[...]
    acc = (xf + mm + sb_ref[...] + bterm
           + w01[:, 0:1] * y0_ref[0] + w01[:, 1:2] * y1_ref[0])
    out_ref[...] = jnp.maximum(acc, 0.0)


def _gating(x, gate_W, gb):
    lt = jnp.tril(jnp.ones((TM, TM), jnp.bfloat16), -1)
    return pl.pallas_call(
        _a1_body,
        grid=(T // TM,),
        in_specs=[
            pl.BlockSpec((TM, D), lambda i: (i, 0)),
            pl.BlockSpec((E, D), lambda i: (0, 0)),
            pl.BlockSpec((1, E), lambda i: (0, 0)),
            pl.BlockSpec((TM, TM), lambda i: (0, 0)),
        ],
        out_specs=[
            pl.BlockSpec((TM, E), lambda i: (i, 0)),
            pl.BlockSpec((TM, 2), lambda i: (i, 0)),
            pl.BlockSpec((TM, 2), lambda i: (i, 0)),
            pl.BlockSpec((TM, 2), lambda i: (i, 0)),
            pl.BlockSpec((1, E), lambda i: (0, 0)),
        ],
        out_shape=[
            jax.ShapeDtypeStruct((T, E), jnp.float32),
            jax.ShapeDtypeStruct((T, 2), jnp.int32),
            jax.ShapeDtypeStruct((T, 2), jnp.float32),
            jax.ShapeDtypeStruct((T, 2), jnp.int32),
            jax.ShapeDtypeStruct((1, E), jnp.float32),
        ],
        scratch_shapes=[pltpu.VMEM((1, E), jnp.float32)],
        compiler_params=pltpu.CompilerParams(
            dimension_semantics=("arbitrary",),
        ),
    )(x, gate_W, gb, lt)


def _positions(counts, e01, r01):
    return pl.pallas_call(
        _a2_body,
        grid=(1,),
        in_specs=[
            pl.BlockSpec((1, E), lambda i: (0, 0)),
            pl.BlockSpec((T, 2), lambda i: (0, 0)),
            pl.BlockSpec((T, 2), lambda i: (0, 0)),
        ],
        out_specs=[
            pl.BlockSpec((T, 2), lambda i: (0, 0)),
            pl.BlockSpec((1, NT), lambda i: (0, 0)),
        ],
        out_shape=[
            jax.ShapeDtypeStruct((T, 2), jnp.int32),
            jax.ShapeDtypeStruct((1, NT), jnp.int32),
        ],
    )(counts, e01, r01)


def _grouped_matmul(xs, W_bf, eid):
    grid_spec = pltpu.PrefetchScalarGridSpec(
        num_scalar_prefetch=1,
        grid=(NT,),
        in_specs=[
            pl.BlockSpec((MB, D), lambda n, eid: (n, 0)),
            pl.BlockSpec((1, D, D), lambda n, eid: (eid[n], 0, 0)),
        ],
        out_specs=pl.BlockSpec((MB, D), lambda n, eid: (n, 0)),
    )
    return pl.pallas_call(
        _c_body,
        grid_spec=grid_spec,
        out_shape=jax.ShapeDtypeStruct((R, D), jnp.float32),
        compiler_params=pltpu.CompilerParams(
            dimension_semantics=("arbitrary",),
        ),
    )(eid, xs, W_bf)


def _combine(x, sw_bf, sb, eb, g, w01, y0, y1):
    return pl.pallas_call(
        _e_body,
        grid=(T // TME,),
        in_specs=[
            pl.BlockSpec((TME, D), lambda i: (i, 0)),
            pl.BlockSpec((D, D), lambda i: (0, 0)),
            pl.BlockSpec((1, D), lambda i: (0, 0)),
            pl.BlockSpec((E, D), lambda i: (0, 0)),
            pl.BlockSpec((TME, E), lambda i: (i, 0)),
            pl.BlockSpec((TME, 2), lambda i: (i, 0)),
            pl.BlockSpec((1, TME, D), lambda i: (0, i, 0)),
            pl.BlockSpec((1, TME, D), lambda i: (1, i, 0)),
        ],
        out_specs=pl.BlockSpec((TME, D), lambda i: (i, 0)),
        out_shape=jax.ShapeDtypeStruct((T, D), jnp.float32),
        compiler_params=pltpu.CompilerParams(
            dimension_semantics=("parallel",),
        ),
    )(x, sw_bf, sb, eb, g, w01, y0, y1)


def kernel(x, shared_W, shared_b, gate_W, gate_b, gate_bias, expert_W, expert_b):
    gb = (gate_b + gate_bias).reshape(1, E)
    g, e01, w01, r01, counts = _gating(x, gate_W, gb)
    pos, eid2 = _positions(counts, e01, r01)
    eid = eid2.reshape(NT)

    pos0_r = pos[:, 0].reshape(NW, NCH, CH)
    pos1_r = pos[:, 1].reshape(NW, NCH, CH)
    xs = _dispatch(x, pos0_r, pos1_r)

    ys = _grouped_matmul(xs, expert_W.astype(jnp.bfloat16), eid)

    posf = jnp.concatenate([pos[:, 0], pos[:, 1]]).reshape(NW, NGC, CH)
    ysg = _collect(ys, posf).reshape(2, T, D)

    return _combine(x, shared_W.astype(jnp.bfloat16), shared_b.reshape(1, D),
                    expert_b, g, w01, ysg, ysg)


# speedup vs baseline: 4.3674x; 1.0055x over previous
import functools

import jax
import jax.numpy as jnp
from jax import lax
from jax.experimental import pallas as pl
from jax.experimental.pallas import tpu as pltpu
from jax.experimental.pallas import tpu_sc as plsc

T = 4096
D = 2048
E = 8
TM = 512
MB = 256
R = 2 * T + E * MB
NT = R // MB
TME = 512

NW = 32
TW = T // NW
CH = 16
NCH = TW // CH
GW = 2 * T // NW
NGC = GW // CH

_MESH = plsc.VectorSubcoreMesh(core_axis_name="c", subcore_axis_name="s")


@functools.partial(
    pl.kernel, mesh=_MESH,
    out_type=jax.ShapeDtypeStruct((R, D), jnp.float32),
    scratch_types=[
        pltpu.VMEM((CH, D), jnp.float32),
        pltpu.VMEM((CH, D), jnp.float32),
        pltpu.VMEM((NCH, CH), jnp.int32),
        pltpu.VMEM((NCH, CH), jnp.int32),
        pltpu.SemaphoreType.DMA,
        pltpu.SemaphoreType.DMA,
    ],
)
def _dispatch(x_hbm, pos0_hbm, pos1_hbm, xs_hbm, xa, xb, p0_v, p1_v,
              ld_sem, st_sem):
    wid = lax.axis_index("s") * 2 + lax.axis_index("c")
    base = wid * TW
    pltpu.sync_copy(pos0_hbm.at[wid], p0_v)
    pltpu.sync_copy(pos1_hbm.at[wid], p1_v)
    bufs = [xa, xb]
    lds = [pltpu.async_copy(x_hbm.at[pl.ds(base, CH)], xa, ld_sem), None]
    scat = []
    for c in range(NCH):
        cur = bufs[c % 2]
        lds[c % 2].wait()
        scat.append(
            (pltpu.async_copy(cur, xs_hbm.at[p0_v.at[c]], st_sem),
             pltpu.async_copy(cur, xs_hbm.at[p1_v.at[c]], st_sem)))
        if c + 1 < NCH:
            if c >= 1:
                scat[c - 1][0].wait()
                scat[c - 1][1].wait()
            lds[(c + 1) % 2] = pltpu.async_copy(
                x_hbm.at[pl.ds(base + (c + 1) * CH, CH)],
                bufs[(c + 1) % 2], ld_sem)
    for s0, s1 in scat[NCH - 2:]:
        s0.wait()
        s1.wait()


@functools.partial(
    pl.kernel, mesh=_MESH,
    out_type=jax.ShapeDtypeStruct((2 * T, D), jnp.float32),
    scratch_types=[
        pltpu.VMEM((CH, D), jnp.float32),
        pltpu.VMEM((CH, D), jnp.float32),
        pltpu.VMEM((NGC, CH), jnp.int32),
        pltpu.SemaphoreType.DMA,
        pltpu.SemaphoreType.DMA,
    ],
)
def _collect(ys_hbm, posf_hbm, out_hbm, ra, rb, idx_v, g_sem, w_sem):
    wid = lax.axis_index("s") * 2 + lax.axis_index("c")
    base = wid * GW
    pltpu.sync_copy(posf_hbm.at[wid], idx_v)
    bufs = [ra, rb]
    gs = [pltpu.async_copy(ys_hbm.at[idx_v.at[0]], ra, g_sem), None]
    wrs = []
    for c in range(NGC):
        cur = bufs[c % 2]
        gs[c % 2].wait()
        wrs.append(pltpu.async_copy(
            cur, out_hbm.at[pl.ds(base + c * CH, CH)], w_sem))
        if c + 1 < NGC:
            if c >= 1:
                wrs[c - 1].wait()
            gs[(c + 1) % 2] = pltpu.async_copy(
                ys_hbm.at[idx_v.at[c + 1]], bufs[(c + 1) % 2], g_sem)
    for w in wrs[NGC - 2:]:
        w.wait()


def _a1_body(x_ref, gw_ref, gb_ref, lt_ref,
             g_ref, e01_ref, w01_ref, r01_ref, cnt_out_ref, cnt_ref):
    i = pl.program_id(0)

    @pl.when(i == 0)
    def _():
        cnt_ref[...] = jnp.zeros((1, E), jnp.float32)

    xf = x_ref[...]
    gs = jax.lax.dot_general(
        xf, gw_ref[...], (((1,), (1,)), ((), ())),
        preferred_element_type=jnp.float32,
    ) + gb_ref[...]
    lanes = jax.lax.broadcasted_iota(jnp.int32, (TM, E), 1)
    i0 = jnp.argmax(gs, axis=1)[:, None]
    oh0 = lanes == i0
    masked = jnp.where(oh0, -jnp.inf, gs)
    i1 = jnp.argmax(masked, axis=1)[:, None]
    oh1 = lanes == i1
    m0 = jnp.max(gs, axis=1)[:, None]
    m1 = jnp.max(masked, axis=1)[:, None]
    e1 = jnp.exp(m1 - m0)
    w0 = 1.0 / (1.0 + e1)
    w1 = e1 / (1.0 + e1)
    g_ref[...] = jnp.where(oh0, w0, 0.0) + jnp.where(oh1, w1, 0.0)
    sel = (oh0 | oh1).astype(jnp.float32)
    rank_local = jax.lax.dot_general(
        lt_ref[...], sel.astype(jnp.bfloat16), (((1,), (0,)), ((), ())),
        preferred_element_type=jnp.float32,
    )
    rank = rank_local + cnt_ref[...]
    cnt_ref[...] += jnp.sum(sel, axis=0, keepdims=True)
    cnt_out_ref[...] = cnt_ref[...]
    e01_ref[...] = jnp.concatenate([i0, i1], axis=1).astype(jnp.int32)
    w01_ref[...] = jnp.concatenate([w0, w1], axis=1)
    r0 = jnp.sum(jnp.where(oh0, rank, 0.0), axis=1, keepdims=True)
    r1 = jnp.sum(jnp.where(oh1, rank, 0.0), axis=1, keepdims=True)
    r01_ref[...] = jnp.concatenate([r0, r1], axis=1).astype(jnp.int32)


def _a2_body(cnt_ref, e01_ref, r01_ref, pos_ref, eid_ref):
    counts = cnt_ref[...]
    npad = jnp.ceil(counts / MB) * MB
    ri = jax.lax.broadcasted_iota(jnp.int32, (E, E), 0)
    ci = jax.lax.broadcasted_iota(jnp.int32, (E, E), 1)
    m = (ri < ci).astype(jnp.float32)
    start = jax.lax.dot_general(
        npad, m, (((1,), (0,)), ((), ())),
        preferred_element_type=jnp.float32,
        precision=jax.lax.Precision.HIGHEST,
    )
    e01 = e01_ref[...]
    base = jnp.zeros(e01.shape, jnp.float32)
    for e in range(E):
        base += jnp.where(e01 == e, start[0, e], 0.0)
    pos_ref[...] = (base + r01_ref[...].astype(jnp.float32)).astype(jnp.int32)
    trow = jax.lax.broadcasted_iota(jnp.int32, (1, NT), 1).astype(jnp.float32) * MB
    eid = jnp.full((1, NT), E - 1, jnp.int32)
    for e in range(E):
        inside = (trow >= start[0, e]) & (trow < start[0, e] + npad[0, e])
        eid = jnp.where(inside, e, eid)
    eid_ref[...] = eid


def _c_body(eid_ref, xs_ref, w_ref, ys_ref):
    ys_ref[...] = jax.lax.dot_general(
        xs_ref[...].astype(jnp.bfloat16), w_ref[0], (((1,), (1,)), ((), ())),
        preferred_element_type=jnp.float32,
    )


def _e_body(x_ref, sw_ref, sb_ref, eb_ref, g_ref, w01_ref, y0_ref, y1_ref,
            out_ref):
    xf = x_ref[...]
    mm = jax.lax.dot_general(
        xf.astype(jnp.bfloat16), sw_ref[...], (((1,), (1,)), ((), ())),
        preferred_element_type=jnp.float32,
    )
    bterm = jax.lax.dot_general(
        g_ref[...], eb_ref[...], (((1,), (0,)), ((), ())),
        preferred_element_type=jnp.float32,
    )
    w01 = w01_ref[...]
    acc = (xf + mm + sb_ref[...] + bterm
           + w01[:, 0:1] * y0_ref[0] + w01[:, 1:2] * y1_ref[0])
    out_ref[...] = jnp.maximum(acc, 0.0)


def _gating(x, gate_W, gb):
    lt = jnp.tril(jnp.ones((TM, TM), jnp.bfloat16), -1)
    return pl.pallas_call(
        _a1_body,
        grid=(T // TM,),
        in_specs=[
            pl.BlockSpec((TM, D), lambda i: (i, 0)),
            pl.BlockSpec((E, D), lambda i: (0, 0)),
            pl.BlockSpec((1, E), lambda i: (0, 0)),
            pl.BlockSpec((TM, TM), lambda i: (0, 0)),
        ],
        out_specs=[
            pl.BlockSpec((TM, E), lambda i: (i, 0)),
            pl.BlockSpec((TM, 2), lambda i: (i, 0)),
            pl.BlockSpec((TM, 2), lambda i: (i, 0)),
            pl.BlockSpec((TM, 2), lambda i: (i, 0)),
            pl.BlockSpec((1, E), lambda i: (0, 0)),
        ],
        out_shape=[
            jax.ShapeDtypeStruct((T, E), jnp.float32),
            jax.ShapeDtypeStruct((T, 2), jnp.int32),
            jax.ShapeDtypeStruct((T, 2), jnp.float32),
            jax.ShapeDtypeStruct((T, 2), jnp.int32),
            jax.ShapeDtypeStruct((1, E), jnp.float32),
        ],
        scratch_shapes=[pltpu.VMEM((1, E), jnp.float32)],
        compiler_params=pltpu.CompilerParams(
            dimension_semantics=("arbitrary",),
        ),
    )(x, gate_W, gb, lt)


def _positions(counts, e01, r01):
    return pl.pallas_call(
        _a2_body,
        grid=(1,),
        in_specs=[
            pl.BlockSpec((1, E), lambda i: (0, 0)),
            pl.BlockSpec((T, 2), lambda i: (0, 0)),
            pl.BlockSpec((T, 2), lambda i: (0, 0)),
        ],
        out_specs=[
            pl.BlockSpec((T, 2), lambda i: (0, 0)),
            pl.BlockSpec((1, NT), lambda i: (0, 0)),
        ],
        out_shape=[
            jax.ShapeDtypeStruct((T, 2), jnp.int32),
            jax.ShapeDtypeStruct((1, NT), jnp.int32),
        ],
    )(counts, e01, r01)


def _grouped_matmul(xs, W_bf, eid):
    grid_spec = pltpu.PrefetchScalarGridSpec(
        num_scalar_prefetch=1,
        grid=(NT,),
        in_specs=[
            pl.BlockSpec((MB, D), lambda n, eid: (n, 0)),
            pl.BlockSpec((1, D, D), lambda n, eid: (eid[n], 0, 0)),
        ],
        out_specs=pl.BlockSpec((MB, D), lambda n, eid: (n, 0)),
    )
    return pl.pallas_call(
        _c_body,
        grid_spec=grid_spec,
        out_shape=jax.ShapeDtypeStruct((R, D), jnp.float32),
        compiler_params=pltpu.CompilerParams(
            dimension_semantics=("arbitrary",),
        ),
    )(eid, xs, W_bf)


def _combine(x, sw_bf, sb, eb, g, w01, y0, y1):
    return pl.pallas_call(
        _e_body,
        grid=(T // TME,),
        in_specs=[
            pl.BlockSpec((TME, D), lambda i: (i, 0)),
            pl.BlockSpec((D, D), lambda i: (0, 0)),
            pl.BlockSpec((1, D), lambda i: (0, 0)),
            pl.BlockSpec((E, D), lambda i: (0, 0)),
            pl.BlockSpec((TME, E), lambda i: (i, 0)),
            pl.BlockSpec((TME, 2), lambda i: (i, 0)),
            pl.BlockSpec((1, TME, D), lambda i: (0, i, 0)),
            pl.BlockSpec((1, TME, D), lambda i: (1, i, 0)),
        ],
        out_specs=pl.BlockSpec((TME, D), lambda i: (i, 0)),
        out_shape=jax.ShapeDtypeStruct((T, D), jnp.float32),
        compiler_params=pltpu.CompilerParams(
            dimension_semantics=("parallel",),
        ),
    )(x, sw_bf, sb, eb, g, w01, y0, y1)


def kernel(x, shared_W, shared_b, gate_W, gate_b, gate_bias, expert_W, expert_b):
    gb = (gate_b + gate_bias).reshape(1, E)
    g, e01, w01, r01, counts = _gating(x, gate_W, gb)
    pos, eid2 = _positions(counts, e01, r01)
    eid = eid2.reshape(NT)

    pos0_r = pos[:, 0].reshape(NW, NCH, CH)
    pos1_r = pos[:, 1].reshape(NW, NCH, CH)
    xs = _dispatch(x, pos0_r, pos1_r)

    ys = _grouped_matmul(xs, expert_W.astype(jnp.bfloat16), eid)

    posf = jnp.concatenate([pos[:, 0], pos[:, 1]]).reshape(NW, NGC, CH)
    ysg = _collect(ys, posf).reshape(2, T, D)

    return _combine(x, shared_W.astype(jnp.bfloat16), shared_b.reshape(1, D),
                    expert_b, g, w01, ysg, ysg)


# speedup vs baseline: 4.3783x; 1.0025x over previous
import functools

import jax
import jax.numpy as jnp
from jax import lax
from jax.experimental import pallas as pl
from jax.experimental.pallas import tpu as pltpu
from jax.experimental.pallas import tpu_sc as plsc

T = 4096
D = 2048
E = 8
TM = 512
MB = 256
R = 2 * T + E * MB
NT = R // MB
TME = 512

NW = 32
TW = T // NW
CH = 16
NCH = TW // CH
GW = 2 * T // NW
NGC = GW // CH

_MESH = plsc.VectorSubcoreMesh(core_axis_name="c", subcore_axis_name="s")


@functools.partial(
    pl.kernel, mesh=_MESH,
    out_type=jax.ShapeDtypeStruct((R, D), jnp.float32),
    scratch_types=[
        pltpu.VMEM((CH, D), jnp.float32),
        pltpu.VMEM((CH, D), jnp.float32),
        pltpu.VMEM((CH, D), jnp.float32),
        pltpu.VMEM((NCH, CH), jnp.int32),
        pltpu.VMEM((NCH, CH), jnp.int32),
        pltpu.SemaphoreType.DMA,
        pltpu.SemaphoreType.DMA,
    ],
)
def _dispatch(x_hbm, pos0_hbm, pos1_hbm, xs_hbm, xa, xb, xc, p0_v, p1_v,
              ld_sem, st_sem):
    wid = lax.axis_index("s") * 2 + lax.axis_index("c")
    base = wid * TW
    pltpu.sync_copy(pos0_hbm.at[wid], p0_v)
    pltpu.sync_copy(pos1_hbm.at[wid], p1_v)
    bufs = [xa, xb, xc]
    lds = {}
    for n in range(2):
        lds[n] = pltpu.async_copy(
            x_hbm.at[pl.ds(base + n * CH, CH)], bufs[n], ld_sem)
    scat = []
    for c in range(NCH):
        cur = bufs[c % 3]
        lds[c].wait()
        scat.append(
            (pltpu.async_copy(cur, xs_hbm.at[p0_v.at[c]], st_sem),
             pltpu.async_copy(cur, xs_hbm.at[p1_v.at[c]], st_sem)))
        n = c + 2
        if n < NCH:
            if c >= 1:
                scat[c - 1][0].wait()
                scat[c - 1][1].wait()
            lds[n] = pltpu.async_copy(
                x_hbm.at[pl.ds(base + n * CH, CH)], bufs[n % 3], ld_sem)
    for s0, s1 in scat[NCH - 3:]:
        s0.wait()
        s1.wait()


@functools.partial(
    pl.kernel, mesh=_MESH,
    out_type=jax.ShapeDtypeStruct((2 * T, D), jnp.float32),
    scratch_types=[
        pltpu.VMEM((CH, D), jnp.float32),
        pltpu.VMEM((CH, D), jnp.float32),
        pltpu.VMEM((CH, D), jnp.float32),
        pltpu.VMEM((NGC, CH), jnp.int32),
        pltpu.SemaphoreType.DMA,
        pltpu.SemaphoreType.DMA,
    ],
)
def _collect(ys_hbm, posf_hbm, out_hbm, ra, rb, rc, idx_v, g_sem, w_sem):
    wid = lax.axis_index("s") * 2 + lax.axis_index("c")
    base = wid * GW
    pltpu.sync_copy(posf_hbm.at[wid], idx_v)
    bufs = [ra, rb, rc]
    gs = {}
    for n in range(2):
        gs[n] = pltpu.async_copy(ys_hbm.at[idx_v.at[n]], bufs[n], g_sem)
    wrs = []
    for c in range(NGC):
        cur = bufs[c % 3]
        gs[c].wait()
        wrs.append(pltpu.async_copy(
            cur, out_hbm.at[pl.ds(base + c * CH, CH)], w_sem))
        n = c + 2
        if n < NGC:
            if c >= 1:
                wrs[c - 1].wait()
            gs[n] = pltpu.async_copy(
                ys_hbm.at[idx_v.at[n]], bufs[n % 3], g_sem)
    for w in wrs[NGC - 3:]:
        w.wait()


def _a1_body(x_ref, gw_ref, gb_ref, lt_ref,
             g_ref, e01_ref, w01_ref, r01_ref, cnt_out_ref, cnt_ref):
    i = pl.program_id(0)

    @pl.when(i == 0)
    def _():
        cnt_ref[...] = jnp.zeros((1, E), jnp.float32)

    xf = x_ref[...]
    gs = jax.lax.dot_general(
        xf, gw_ref[...], (((1,), (1,)), ((), ())),
        preferred_element_type=jnp.float32,
    ) + gb_ref[...]
    lanes = jax.lax.broadcasted_iota(jnp.int32, (TM, E), 1)
    i0 = jnp.argmax(gs, axis=1)[:, None]
    oh0 = lanes == i0
    masked = jnp.where(oh0, -jnp.inf, gs)
    i1 = jnp.argmax(masked, axis=1)[:, None]
    oh1 = lanes == i1
    m0 = jnp.max(gs, axis=1)[:, None]
    m1 = jnp.max(masked, axis=1)[:, None]
    e1 = jnp.exp(m1 - m0)
    w0 = 1.0 / (1.0 + e1)
    w1 = e1 / (1.0 + e1)
    g_ref[...] = jnp.where(oh0, w0, 0.0) + jnp.where(oh1, w1, 0.0)
    sel = (oh0 | oh1).astype(jnp.float32)
    rank_local = jax.lax.dot_general(
        lt_ref[...], sel.astype(jnp.bfloat16), (((1,), (0,)), ((), ())),
        preferred_element_type=jnp.float32,
    )
    rank = rank_local + cnt_ref[...]
    cnt_ref[...] += jnp.sum(sel, axis=0, keepdims=True)
    cnt_out_ref[...] = cnt_ref[...]
    e01_ref[...] = jnp.concatenate([i0, i1], axis=1).astype(jnp.int32)
    w01_ref[...] = jnp.concatenate([w0, w1], axis=1)
    r0 = jnp.sum(jnp.where(oh0, rank, 0.0), axis=1, keepdims=True)
    r1 = jnp.sum(jnp.where(oh1, rank, 0.0), axis=1, keepdims=True)
    r01_ref[...] = jnp.concatenate([r0, r1], axis=1).astype(jnp.int32)


def _a2_body(cnt_ref, e01_ref, r01_ref, pos_ref, eid_ref):
    counts = cnt_ref[...]
    npad = jnp.ceil(counts / MB) * MB
    ri = jax.lax.broadcasted_iota(jnp.int32, (E, E), 0)
    ci = jax.lax.broadcasted_iota(jnp.int32, (E, E), 1)
    m = (ri < ci).astype(jnp.float32)
    start = jax.lax.dot_general(
        npad, m, (((1,), (0,)), ((), ())),
        preferred_element_type=jnp.float32,
        precision=jax.lax.Precision.HIGHEST,
    )
    e01 = e01_ref[...]
    base = jnp.zeros(e01.shape, jnp.float32)
    for e in range(E):
        base += jnp.where(e01 == e, start[0, e], 0.0)
    pos_ref[...] = (base + r01_ref[...].astype(jnp.float32)).astype(jnp.int32)
    trow = jax.lax.broadcasted_iota(jnp.int32, (1, NT), 1).astype(jnp.float32) * MB
    eid = jnp.full((1, NT), E - 1, jnp.int32)
    for e in range(E):
        inside = (trow >= start[0, e]) & (trow < start[0, e] + npad[0, e])
        eid = jnp.where(inside, e, eid)
    eid_ref[...] = eid


def _c_body(eid_ref, xs_ref, w_ref, ys_ref):
    ys_ref[...] = jax.lax.dot_general(
        xs_ref[...].astype(jnp.bfloat16), w_ref[0], (((1,), (1,)), ((), ())),
        preferred_element_type=jnp.float32,
    )


def _e_body(x_ref, sw_ref, sb_ref, eb_ref, g_ref, w01_ref, y0_ref, y1_ref,
            out_ref):
    xf = x_ref[...]
    mm = jax.lax.dot_general(
        xf.astype(jnp.bfloat16), sw_ref[...], (((1,), (1,)), ((), ())),
        preferred_element_type=jnp.float32,
    )
    bterm = jax.lax.dot_general(
        g_ref[...], eb_ref[...], (((1,), (0,)), ((), ())),
        preferred_element_type=jnp.float32,
    )
    w01 = w01_ref[...]
    acc = (xf + mm + sb_ref[...] + bterm
           + w01[:, 0:1] * y0_ref[0] + w01[:, 1:2] * y1_ref[0])
    out_ref[...] = jnp.maximum(acc, 0.0)


def _gating(x, gate_W, gb):
    lt = jnp.tril(jnp.ones((TM, TM), jnp.bfloat16), -1)
    return pl.pallas_call(
        _a1_body,
        grid=(T // TM,),
        in_specs=[
            pl.BlockSpec((TM, D), lambda i: (i, 0)),
            pl.BlockSpec((E, D), lambda i: (0, 0)),
            pl.BlockSpec((1, E), lambda i: (0, 0)),
            pl.BlockSpec((TM, TM), lambda i: (0, 0)),
        ],
        out_specs=[
            pl.BlockSpec((TM, E), lambda i: (i, 0)),
            pl.BlockSpec((TM, 2), lambda i: (i, 0)),
            pl.BlockSpec((TM, 2), lambda i: (i, 0)),
            pl.BlockSpec((TM, 2), lambda i: (i, 0)),
            pl.BlockSpec((1, E), lambda i: (0, 0)),
        ],
        out_shape=[
            jax.ShapeDtypeStruct((T, E), jnp.float32),
            jax.ShapeDtypeStruct((T, 2), jnp.int32),
            jax.ShapeDtypeStruct((T, 2), jnp.float32),
            jax.ShapeDtypeStruct((T, 2), jnp.int32),
            jax.ShapeDtypeStruct((1, E), jnp.float32),
        ],
        scratch_shapes=[pltpu.VMEM((1, E), jnp.float32)],
        compiler_params=pltpu.CompilerParams(
            dimension_semantics=("arbitrary",),
        ),
    )(x, gate_W, gb, lt)


def _positions(counts, e01, r01):
    return pl.pallas_call(
        _a2_body,
        grid=(1,),
        in_specs=[
            pl.BlockSpec((1, E), lambda i: (0, 0)),
            pl.BlockSpec((T, 2), lambda i: (0, 0)),
            pl.BlockSpec((T, 2), lambda i: (0, 0)),
        ],
        out_specs=[
            pl.BlockSpec((T, 2), lambda i: (0, 0)),
            pl.BlockSpec((1, NT), lambda i: (0, 0)),
        ],
        out_shape=[
            jax.ShapeDtypeStruct((T, 2), jnp.int32),
            jax.ShapeDtypeStruct((1, NT), jnp.int32),
        ],
    )(counts, e01, r01)


def _grouped_matmul(xs, W_bf, eid):
    grid_spec = pltpu.PrefetchScalarGridSpec(
        num_scalar_prefetch=1,
        grid=(NT,),
        in_specs=[
            pl.BlockSpec((MB, D), lambda n, eid: (n, 0)),
            pl.BlockSpec((1, D, D), lambda n, eid: (eid[n], 0, 0)),
        ],
        out_specs=pl.BlockSpec((MB, D), lambda n, eid: (n, 0)),
    )
    return pl.pallas_call(
        _c_body,
        grid_spec=grid_spec,
        out_shape=jax.ShapeDtypeStruct((R, D), jnp.float32),
        compiler_params=pltpu.CompilerParams(
            dimension_semantics=("arbitrary",),
        ),
    )(eid, xs, W_bf)


def _combine(x, sw_bf, sb, eb, g, w01, y0, y1):
    return pl.pallas_call(
        _e_body,
        grid=(T // TME,),
        in_specs=[
            pl.BlockSpec((TME, D), lambda i: (i, 0)),
            pl.BlockSpec((D, D), lambda i: (0, 0)),
            pl.BlockSpec((1, D), lambda i: (0, 0)),
            pl.BlockSpec((E, D), lambda i: (0, 0)),
            pl.BlockSpec((TME, E), lambda i: (i, 0)),
            pl.BlockSpec((TME, 2), lambda i: (i, 0)),
            pl.BlockSpec((1, TME, D), lambda i: (0, i, 0)),
            pl.BlockSpec((1, TME, D), lambda i: (1, i, 0)),
        ],
        out_specs=pl.BlockSpec((TME, D), lambda i: (i, 0)),
        out_shape=jax.ShapeDtypeStruct((T, D), jnp.float32),
        compiler_params=pltpu.CompilerParams(
            dimension_semantics=("parallel",),
        ),
    )(x, sw_bf, sb, eb, g, w01, y0, y1)


def kernel(x, shared_W, shared_b, gate_W, gate_b, gate_bias, expert_W, expert_b):
    gb = (gate_b + gate_bias).reshape(1, E)
    g, e01, w01, r01, counts = _gating(x, gate_W, gb)
    pos, eid2 = _positions(counts, e01, r01)
    eid = eid2.reshape(NT)

    pos0_r = pos[:, 0].reshape(NW, NCH, CH)
    pos1_r = pos[:, 1].reshape(NW, NCH, CH)
    xs = _dispatch(x, pos0_r, pos1_r)

    ys = _grouped_matmul(xs, expert_W.astype(jnp.bfloat16), eid)

    posf = jnp.concatenate([pos[:, 0], pos[:, 1]]).reshape(NW, NGC, CH)
    ysg = _collect(ys, posf).reshape(2, T, D)

    return _combine(x, shared_W.astype(jnp.bfloat16), shared_b.reshape(1, D),
                    expert_b, g, w01, ysg, ysg)
